# baseline (speedup 1.0000x reference)
import functools
import math

import jax
import jax.numpy as jnp
from jax import lax
from jax.experimental import pallas as pl
from jax.experimental.pallas import tpu as pltpu
from jax.experimental.pallas import tpu_sc as plsc

N_EMB = 64
SCALE = math.sqrt(N_EMB)

NC = 2
NS = 16
NW = NC * NS

NBUF = 2
LOOKAHEAD = 1


def _embed_call(xf, lut128, R, C):
    D = N_EMB
    r_per_w = R // NW
    b_per_w = r_per_w * C
    mesh = plsc.VectorSubcoreMesh(core_axis_name="c", subcore_axis_name="s")

    @functools.partial(
        pl.kernel,
        out_type=jax.ShapeDtypeStruct((R, C, 128), jnp.float32),
        mesh=mesh,
        scratch_types=[
            pltpu.VMEM((b_per_w,), jnp.int32),
            [pltpu.VMEM((C, 128), jnp.float32) for _ in range(NBUF)],
            [pltpu.SemaphoreType.DMA for _ in range(NBUF)],
            [pltpu.SemaphoreType.DMA for _ in range(NBUF)],
        ],
        compiler_params=pltpu.CompilerParams(use_tc_tiling_on_sc=True),
    )
    def k(x_hbm, lut_hbm, out_hbm, idx_v, rows, gsem, ssem):
        wid = lax.axis_index("s") * NC + lax.axis_index("c")
        base = wid * b_per_w
        row0 = wid * r_per_w

        pltpu.sync_copy(x_hbm.at[pl.ds(base, b_per_w)], idx_v)

        def start_gather(c, b):
            pltpu.async_copy(
                lut_hbm.at[idx_v.at[pl.ds(c * C, C)]], rows[b], gsem[b])

        def wait_gather(c, b):
            pltpu.make_async_copy(
                lut_hbm.at[idx_v.at[pl.ds(c * C, C)]], rows[b], gsem[b]).wait()

        def start_scatter(c, b):
            pltpu.async_copy(rows[b], out_hbm.at[row0 + c], ssem[b])

        def wait_scatter(c, b):
            pltpu.make_async_copy(
                rows[b], out_hbm.at[row0 + c], ssem[b]).wait()

        for j in range(LOOKAHEAD):
            start_gather(j, j)

        def outer(io, carry):
            for b in range(NBUF):
                i = io * NBUF + b
                nb = (b + LOOKAHEAD) % NBUF

                @pl.when(i + LOOKAHEAD < r_per_w)
                def _():
                    @pl.when(i + LOOKAHEAD >= NBUF)
                    def _():
                        wait_scatter(i + LOOKAHEAD - NBUF, nb)
                    start_gather(i + LOOKAHEAD, nb)

                wait_gather(i, b)

                @plsc.parallel_loop(0, C, step=1, unroll=8)
                def _(r):
                    for j in range(D // 16):
                        sl = pl.ds(j * 16, 16)
                        rows[b][r, sl] = rows[b][r, sl] * SCALE

                start_scatter(i, b)
            return carry

        lax.fori_loop(0, r_per_w // NBUF, outer, 0)

        for b in range(NBUF):
            wait_scatter(r_per_w - NBUF + b, b)

    return k(xf, lut128)


def kernel(x, lut):
    R, C = x.shape
    xf = x.reshape(R * C).astype(jnp.int32)
    lut128 = jnp.pad(lut, ((0, 0), (0, 128 - N_EMB)))
    return _embed_call(xf, lut128, R, C)[:, :, :N_EMB]


# speedup vs baseline: 1.0924x; 1.0924x over previous
import functools
import math

import jax
import jax.numpy as jnp
from jax import lax
from jax.experimental import pallas as pl
from jax.experimental.pallas import tpu as pltpu
from jax.experimental.pallas import tpu_sc as plsc

N_EMB = 64
SCALE = math.sqrt(N_EMB)

NC = 2
NS = 16
NW = NC * NS

NBUF = 4
LOOKAHEAD = 2


def _embed_call(xf, lut, R, C):
    D = N_EMB
    r_per_w = R // NW
    b_per_w = r_per_w * C
    mesh = plsc.VectorSubcoreMesh(core_axis_name="c", subcore_axis_name="s")

    @functools.partial(
        pl.kernel,
        out_type=jax.ShapeDtypeStruct((R, C, 128), jnp.float32),
        mesh=mesh,
        scratch_types=[
            pltpu.VMEM((b_per_w,), jnp.int32),
            [pltpu.VMEM((C, D), jnp.float32) for _ in range(NBUF)],
            [pltpu.SemaphoreType.DMA for _ in range(NBUF)],
            [pltpu.SemaphoreType.DMA for _ in range(NBUF)],
        ],
        compiler_params=pltpu.CompilerParams(use_tc_tiling_on_sc=False),
    )
    def k(x_hbm, lut_hbm, out_hbm, idx_v, rows, gsem, ssem):
        wid = lax.axis_index("s") * NC + lax.axis_index("c")
        base = wid * b_per_w
        row0 = wid * r_per_w

        pltpu.sync_copy(x_hbm.at[pl.ds(base, b_per_w)], idx_v)

        def start_gather(c, b):
            pltpu.async_copy(
                lut_hbm.at[idx_v.at[pl.ds(c * C, C)]], rows[b], gsem[b])

        def wait_gather(c, b):
            pltpu.make_async_copy(
                lut_hbm.at[idx_v.at[pl.ds(c * C, C)]], rows[b], gsem[b]).wait()

        def start_scatter(c, b):
            pltpu.async_copy(
                rows[b], out_hbm.at[row0 + c, :, pl.ds(0, D)], ssem[b])

        def wait_scatter(c, b):
            pltpu.make_async_copy(
                rows[b], out_hbm.at[row0 + c, :, pl.ds(0, D)], ssem[b]).wait()

        for j in range(LOOKAHEAD):
            start_gather(j, j)

        def outer(io, carry):
            for b in range(NBUF):
                i = io * NBUF + b
                nb = (b + LOOKAHEAD) % NBUF

                @pl.when(i + LOOKAHEAD < r_per_w)
                def _():
                    @pl.when(i + LOOKAHEAD >= NBUF)
                    def _():
                        wait_scatter(i + LOOKAHEAD - NBUF, nb)
                    start_gather(i + LOOKAHEAD, nb)

                wait_gather(i, b)

                @plsc.parallel_loop(0, C, step=1, unroll=8)
                def _(r):
                    for j in range(D // 16):
                        sl = pl.ds(j * 16, 16)
                        rows[b][r, sl] = rows[b][r, sl] * SCALE

                start_scatter(i, b)
            return carry

        lax.fori_loop(0, r_per_w // NBUF, outer, 0)

        for b in range(NBUF):
            wait_scatter(r_per_w - NBUF + b, b)

    return k(xf, lut)


def kernel(x, lut):
    R, C = x.shape
    xf = x.reshape(R * C).astype(jnp.int32)
    return _embed_call(xf, lut, R, C)[:, :, :N_EMB]
